# baseline (speedup 1.0000x reference)
import jax
import jax.numpy as jnp
from jax import lax
from jax.experimental import pallas as pl

INPUT_DIM = 1024
LATENT_DIM = 64
NUM_EMBEDDINGS = 1024
BATCH = 16384

TILE = 512
HALF = BATCH // 2


def _e2_body(emb_ref, e2_ref):
    e2_ref[...] = jnp.sum(emb_ref[...] ** 2, axis=0, keepdims=True)


def _vq_body(x_ref, we_ref, be_ref, emb_ref, e2_ref, wd_ref, bd_ref, out_ref):
    x = x_ref[...].reshape(2 * TILE, INPUT_DIM)
    z = jnp.dot(x, we_ref[...], preferred_element_type=jnp.float32) + be_ref[...]
    sim = jnp.dot(z, emb_ref[...], preferred_element_type=jnp.float32)
    d = jnp.sum(z * z, axis=1, keepdims=True) + e2_ref[...] - 2.0 * sim
    idx = jnp.argmin(d, axis=1)
    enc = (lax.broadcasted_iota(jnp.int32, (2 * TILE, NUM_EMBEDDINGS), 1)
           == idx[:, None]).astype(jnp.float32)
    q = lax.dot_general(enc, emb_ref[...], (((1,), (1,)), ((), ())),
                        preferred_element_type=jnp.float32)
    out = (jnp.dot(q, wd_ref[...], preferred_element_type=jnp.float32)
           + bd_ref[...])
    out_ref[...] = out.reshape(2, TILE, INPUT_DIM)


@jax.jit
def kernel(x, W_enc, b_enc, W_emb, W_dec, b_dec):
    nb = HALF // TILE
    full = lambda shape: pl.BlockSpec(shape, lambda i: (0,) * len(shape))
    e2 = pl.pallas_call(
        _e2_body,
        in_specs=[pl.BlockSpec((LATENT_DIM, NUM_EMBEDDINGS), lambda: (0, 0))],
        out_specs=pl.BlockSpec((1, NUM_EMBEDDINGS), lambda: (0, 0)),
        out_shape=jax.ShapeDtypeStruct((1, NUM_EMBEDDINGS), jnp.float32),
    )(W_emb)
    out = pl.pallas_call(
        _vq_body,
        grid=(nb,),
        in_specs=[
            pl.BlockSpec((2, TILE, INPUT_DIM), lambda i: (0, i, 0)),
            full((INPUT_DIM, LATENT_DIM)),
            full((1, LATENT_DIM)),
            full((LATENT_DIM, NUM_EMBEDDINGS)),
            full((1, NUM_EMBEDDINGS)),
            full((LATENT_DIM, INPUT_DIM)),
            full((1, INPUT_DIM)),
        ],
        out_specs=pl.BlockSpec((2, TILE, INPUT_DIM), lambda i: (0, i, 0)),
        out_shape=jax.ShapeDtypeStruct((2, HALF, INPUT_DIM), jnp.float32),
    )(x.reshape(2, HALF, INPUT_DIM), W_enc, b_enc.reshape(1, -1), W_emb, e2,
      W_dec, b_dec.reshape(1, -1))
    return out.reshape(BATCH, INPUT_DIM)
